# baseline (speedup 1.0000x reference)
import functools

import jax
import jax.numpy as jnp
from jax import lax
from jax.experimental import pallas as pl
from jax.experimental.pallas import tpu as pltpu
from jax.experimental.pallas import tpu_sc as plsc

B = 16384
CTX = 10
D = 64
NC = 2
NS = 16
NW = NC * NS
EPW = B // NW
CH = 64
NCHUNK = EPW // CH
RPC = CH * CTX
GSZ = 128
NG = RPC // GSZ
EPG = 8
NGRP = CH // EPG


def _sc_logits(batch, pos_flat, neg_flat, in_emb, out_emb):
  mesh = plsc.VectorSubcoreMesh(core_axis_name="c", subcore_axis_name="s",
                                num_cores=NC, num_subcores=NS)

  @functools.partial(
      pl.kernel,
      out_type=(jax.ShapeDtypeStruct((B * CTX,), jnp.float32),
                jax.ShapeDtypeStruct((B * CTX,), jnp.float32)),
      mesh=mesh,
      compiler_params=pltpu.CompilerParams(
          needs_layout_passes=False, use_tc_tiling_on_sc=False),
      scratch_types=[
          pltpu.VMEM((CH,), jnp.int32),
          pltpu.VMEM((RPC,), jnp.int32),
          pltpu.VMEM((CH, D), jnp.float32),
          pltpu.VMEM((RPC, D), jnp.float32),
          pltpu.VMEM((EPW * CTX,), jnp.float32),
          pltpu.VMEM((EPW * CTX,), jnp.float32),
          pltpu.SemaphoreType.DMA,
      ],
  )
  def k(batch_hbm, pos_hbm, neg_hbm, in_hbm, out_hbm, opos_hbm, oneg_hbm,
        bidx_v, sidx_v, u_v, rows_v, plog_v, nlog_v, sem):
    wid = lax.axis_index("s") * NC + lax.axis_index("c")
    lidx = lax.iota(jnp.int32, 16)

    def chunk_body(c, _):
      bbase = pl.multiple_of(wid * EPW + c * CH, CH)
      pltpu.sync_copy(batch_hbm.at[pl.ds(bbase, CH)], bidx_v)
      pltpu.async_copy(in_hbm.at[bidx_v], u_v, sem).wait()

      for t in (0, 1):
        samp_hbm = pos_hbm if t == 0 else neg_hbm
        log_v = plog_v if t == 0 else nlog_v
        sign = 1.0 if t == 0 else -1.0
        sbase = pl.multiple_of(wid * (EPW * CTX) + c * RPC, RPC)
        pltpu.sync_copy(samp_hbm.at[pl.ds(sbase, RPC)], sidx_v)
        cps = [
            pltpu.async_copy(
                out_hbm.at[sidx_v.at[pl.ds(g * GSZ, GSZ)]],
                rows_v.at[pl.ds(g * GSZ, GSZ)], sem)
            for g in range(NG)
        ]
        for cp in cps:
          cp.wait()

        def group_body(rg, _, log_v=log_v, sign=sign, c=c):
          base_e = rg * EPG
          svec = jnp.zeros((16,), jnp.float32)
          for e in range(EPG):
            i = base_e + e
            u = [u_v[i, pl.ds(kk * 16, 16)] for kk in range(4)]
            for p in range(CTX):
              r = e * CTX + p
              acc = rows_v[i * CTX + p, pl.ds(0, 16)] * u[0]
              for kk in range(1, 4):
                acc = acc + rows_v[i * CTX + p, pl.ds(kk * 16, 16)] * u[kk]
              s = sign * jnp.sum(acc)
              svec = jnp.where(lidx == (r % 16), s, svec)
              if r % 16 == 15:
                off = pl.multiple_of(
                    c * RPC + rg * (EPG * CTX) + (r - 15), 16)
                log_v[pl.ds(off, 16)] = svec
          return 0

        lax.fori_loop(0, NGRP, group_body, 0)
      return 0

    lax.fori_loop(0, NCHUNK, chunk_body, 0)
    obase = pl.multiple_of(wid * (EPW * CTX), EPW * CTX)
    pltpu.sync_copy(plog_v, opos_hbm.at[pl.ds(obase, EPW * CTX)])
    pltpu.sync_copy(nlog_v, oneg_hbm.at[pl.ds(obase, EPW * CTX)])

  return k(batch, pos_flat, neg_flat, in_emb, out_emb)


def _tc_body(p_ref, n_ref, o_ref):
  def log_sigmoid(x):
    return jnp.minimum(x, 0.0) - jnp.log1p(jnp.exp(-jnp.abs(x)))
  acc = jnp.sum(log_sigmoid(p_ref[...]), axis=1)
  acc = acc + jnp.sum(log_sigmoid(n_ref[...]), axis=1)
  o_ref[...] = -acc


def kernel(batch, pos_samps, neg_samps, in_emb, out_emb):
  batch = batch.astype(jnp.int32)
  pos_flat = pos_samps.astype(jnp.int32).reshape(-1)
  neg_flat = neg_samps.astype(jnp.int32).reshape(-1)

  plog, nlog = _sc_logits(batch, pos_flat, neg_flat, in_emb, out_emb)

  return pl.pallas_call(
      _tc_body,
      out_shape=jax.ShapeDtypeStruct((B,), jnp.float32),
  )(plog.reshape(B, CTX), nlog.reshape(B, CTX))
